# Initial kernel scaffold; baseline (speedup 1.0000x reference)
import jax
import jax.numpy as jnp
from jax.experimental import pallas as pl


def kernel(x, edge_index, W1, att_src1, att_dst1, b1, W2, att_src2, att_dst2, b2):
    raise NotImplementedError("write your pallas kernel here")


# speedup vs baseline: 26.4898x; 26.4898x over previous
import functools

import jax
import jax.numpy as jnp
from jax import lax
from jax.experimental import pallas as pl
from jax.experimental.pallas import tpu as pltpu
from jax.experimental.pallas import tpu_sc as plsc

N = 10000
NPAD = 10240
E = 320000
D = 128
NC = 2
NS = 16
NW = NC * NS
C = 128
NCHUNK = E // C
KMAX = -(-NCHUNK // NW)
RPT = NPAD // NS
BN = 400
GRID = N // BN


def _tc_in_body(x_ref, w_ref, as_ref, ad_ref, h_ref, asrc_ref, adst_ref):
    h = jnp.dot(x_ref[...], w_ref[...], preferred_element_type=jnp.float32)
    h_ref[...] = h
    asrc_ref[...] = jnp.dot(h, as_ref[...], preferred_element_type=jnp.float32)
    adst_ref[...] = jnp.dot(h, ad_ref[...], preferred_element_type=jnp.float32)


def _tc_mid_body(np_ref, dp_ref, b1_ref, w_ref, as_ref, ad_ref,
                 h_ref, asrc_ref, adst_ref):
    num = np_ref[0] + np_ref[1]
    den = dp_ref[0] + dp_ref[1]
    h1 = jnp.maximum(num / (den + 1e-16) + b1_ref[...], 0.0)
    h2 = jnp.dot(h1, w_ref[...], preferred_element_type=jnp.float32)
    h_ref[...] = h2
    asrc_ref[...] = jnp.dot(h2, as_ref[...], preferred_element_type=jnp.float32)
    adst_ref[...] = jnp.dot(h2, ad_ref[...], preferred_element_type=jnp.float32)


def _tc_out_body(np_ref, dp_ref, b2_ref, out_ref):
    num = np_ref[0] + np_ref[1]
    den = dp_ref[0] + dp_ref[1]
    out_ref[...] = num / (den + 1e-16) + b2_ref[...]


_tc_in = pl.pallas_call(
    _tc_in_body,
    grid=(GRID,),
    in_specs=[
        pl.BlockSpec((BN, D), lambda i: (i, 0)),
        pl.BlockSpec((D, D), lambda i: (0, 0)),
        pl.BlockSpec((D, 1), lambda i: (0, 0)),
        pl.BlockSpec((D, 1), lambda i: (0, 0)),
    ],
    out_specs=[
        pl.BlockSpec((BN, D), lambda i: (i, 0)),
        pl.BlockSpec((BN, 1), lambda i: (i, 0)),
        pl.BlockSpec((BN, 1), lambda i: (i, 0)),
    ],
    out_shape=[
        jax.ShapeDtypeStruct((N, D), jnp.float32),
        jax.ShapeDtypeStruct((N, 1), jnp.float32),
        jax.ShapeDtypeStruct((N, 1), jnp.float32),
    ],
)

_tc_mid = pl.pallas_call(
    _tc_mid_body,
    grid=(GRID,),
    in_specs=[
        pl.BlockSpec((2, BN, D), lambda i: (0, i, 0)),
        pl.BlockSpec((2, BN, 1), lambda i: (0, i, 0)),
        pl.BlockSpec((1, D), lambda i: (0, 0)),
        pl.BlockSpec((D, D), lambda i: (0, 0)),
        pl.BlockSpec((D, 1), lambda i: (0, 0)),
        pl.BlockSpec((D, 1), lambda i: (0, 0)),
    ],
    out_specs=[
        pl.BlockSpec((BN, D), lambda i: (i, 0)),
        pl.BlockSpec((BN, 1), lambda i: (i, 0)),
        pl.BlockSpec((BN, 1), lambda i: (i, 0)),
    ],
    out_shape=[
        jax.ShapeDtypeStruct((N, D), jnp.float32),
        jax.ShapeDtypeStruct((N, 1), jnp.float32),
        jax.ShapeDtypeStruct((N, 1), jnp.float32),
    ],
)

_tc_out = pl.pallas_call(
    _tc_out_body,
    grid=(GRID,),
    in_specs=[
        pl.BlockSpec((2, BN, D), lambda i: (0, i, 0)),
        pl.BlockSpec((2, BN, 1), lambda i: (0, i, 0)),
        pl.BlockSpec((1, D), lambda i: (0, 0)),
    ],
    out_specs=pl.BlockSpec((BN, D), lambda i: (i, 0)),
    out_shape=jax.ShapeDtypeStruct((N, D), jnp.float32),
)


def _sc_body(h_hbm, asrc_hbm, adst_hbm, src_hbm, dst_hbm,
             num_hbm, den_hbm,
             acc, accd,
             asrc_t, adst_t, src_v, dst_v, w_v, rows_v, zd_v,
             sem):
    cid = lax.axis_index("c")
    sid = lax.axis_index("s")
    wkr = sid * NC + cid

    pltpu.sync_copy(asrc_hbm, asrc_t)
    pltpu.sync_copy(adst_hbm, adst_t)

    zeros16 = jnp.zeros((16,), jnp.float32)

    def _zero_row(i, _):
        for d in range(D // 16):
            rows_v[i, pl.ds(16 * d, 16)] = zeros16
        return 0

    lax.fori_loop(0, C, _zero_row, 0)

    def _zero_zd(i, _):
        zd_v[pl.ds(16 * i, 16)] = zeros16
        return 0

    lax.fori_loop(0, RPT // 16, _zero_zd, 0)

    for cpy in range(RPT // C):
        pltpu.sync_copy(rows_v, acc.at[pl.ds(sid * RPT + cpy * C, C)])
    pltpu.sync_copy(zd_v, accd.at[pl.ds(sid * RPT, RPT)])
    plsc.subcore_barrier()

    def _chunk(k, _):
        chunk_id = k * NW + wkr

        @pl.when(chunk_id < NCHUNK)
        def _():
            base = chunk_id * C
            pltpu.sync_copy(src_hbm.at[pl.ds(base, C)], src_v)
            pltpu.sync_copy(dst_hbm.at[pl.ds(base, C)], dst_v)
            gdesc = pltpu.async_copy(h_hbm.at[src_v], rows_v, sem)
            for j in range(C // 16):
                si = src_v[pl.ds(16 * j, 16)]
                di = dst_v[pl.ds(16 * j, 16)]
                z = plsc.load_gather(asrc_t, [si]) + plsc.load_gather(adst_t, [di])
                e = jnp.where(z >= 0.0, z, z * 0.2)
                w_v[pl.ds(16 * j, 16)] = jnp.exp(e)
            gdesc.wait()

            def _scale(j, _):
                wv = w_v[pl.ds(16 * j, 16)]
                for e in range(16):
                    wb = jnp.full((16,), wv[e], jnp.float32)
                    row = 16 * j + e
                    for d in range(D // 16):
                        sl = pl.ds(16 * d, 16)
                        rows_v[row, sl] = rows_v[row, sl] * wb
                return 0

            lax.fori_loop(0, C // 16, _scale, 0)

            pltpu.sync_copy(rows_v, acc.at[dst_v], add=True)
            pltpu.sync_copy(w_v, accd.at[dst_v], add=True)

        return 0

    lax.fori_loop(0, KMAX, _chunk, 0)
    plsc.subcore_barrier()

    for cpy in range(RPT // C):
        r = sid * RPT + cpy * C
        pltpu.sync_copy(acc.at[pl.ds(r, C)], rows_v)
        pltpu.sync_copy(rows_v, num_hbm.at[cid, pl.ds(r, C)])
    pltpu.sync_copy(accd.at[pl.ds(sid * RPT, RPT)], zd_v)
    pltpu.sync_copy(zd_v, den_hbm.at[pl.ds(cid * NPAD + sid * RPT, RPT)])


_sc_edge = functools.partial(
    pl.kernel,
    out_type=[
        jax.ShapeDtypeStruct((NC, NPAD, D), jnp.float32),
        jax.ShapeDtypeStruct((NC * NPAD,), jnp.float32),
    ],
    mesh=plsc.VectorSubcoreMesh(
        core_axis_name="c", subcore_axis_name="s",
        num_cores=NC, num_subcores=NS),
    compiler_params=pltpu.CompilerParams(use_tc_tiling_on_sc=False, needs_layout_passes=False),
    scratch_types=[
        pltpu.VMEM_SHARED((NPAD, D), jnp.float32),
        pltpu.VMEM_SHARED((NPAD,), jnp.float32),
        pltpu.VMEM((N,), jnp.float32),
        pltpu.VMEM((N,), jnp.float32),
        pltpu.VMEM((C,), jnp.int32),
        pltpu.VMEM((C,), jnp.int32),
        pltpu.VMEM((C,), jnp.float32),
        pltpu.VMEM((C, D), jnp.float32),
        pltpu.VMEM((RPT,), jnp.float32),
        pltpu.SemaphoreType.DMA,
    ],
)(_sc_body)


def kernel(x, edge_index, W1, att_src1, att_dst1, b1, W2, att_src2, att_dst2, b2):
    src = edge_index[0].astype(jnp.int32)
    dst = edge_index[1].astype(jnp.int32)

    h1, asrc1, adst1 = _tc_in(
        x, W1, att_src1.reshape(D, 1), att_dst1.reshape(D, 1))
    n1, d1 = _sc_edge(h1, asrc1.reshape(N), adst1.reshape(N), src, dst)
    h2, asrc2, adst2 = _tc_mid(
        n1, d1.reshape(NC, NPAD, 1), b1.reshape(1, D), W2,
        att_src2.reshape(D, 1), att_dst2.reshape(D, 1))
    n2, d2 = _sc_edge(h2, asrc2.reshape(N), adst2.reshape(N), src, dst)
    return _tc_out(n2, d2.reshape(NC, NPAD, 1), b2.reshape(1, D))
